# Initial kernel scaffold; baseline (speedup 1.0000x reference)
import functools

import jax
import jax.numpy as jnp
from jax import lax
from jax.experimental import pallas as pl
from jax.experimental.pallas import tpu as pltpu
from jax.experimental.pallas import tpu_sc as plsc

X0 = -51.2
Y0 = -51.2
VS = 0.2
GH = 512
GW = 512
C_OUT = 64
EPS = 1e-3

NPTS = 100000
NS = 16
NC = 2
PT = 6256
PPAD = NS * PT
FW = 80
ZR = 16384
NZ = 32
ACC_R = NS * 1152
DUMP = ZR
BIG = 1 << 29
G = 128
CHUNKS = PT // 16
MAXG = PT // G + 1
SELN = (MAXG + 1) * G


def _mlp_body(pts_ref, w1_ref, b1_ref, w2_ref, b2_ref, feat_ref, idx_ref):
    pid = pl.program_id(0)
    off = jnp.where(pid >= NS, GH * GW, 0)
    pts = pts_ref[0]
    x = pts[0:1, :]
    y = pts[1:2, :]
    gx = ((x - X0) / VS).astype(jnp.int32)
    gy = ((y - Y0) / VS).astype(jnp.int32)
    valid = (gx >= 0) & (gx < GW) & (gy >= 0) & (gy < GH)
    pos = lax.broadcasted_iota(jnp.int32, (1, PT), 1) + (pid % NS) * PT
    valid = valid & (pos < NPTS)
    gxc = jnp.clip(gx, 0, GW - 1)
    gyc = jnp.clip(gy, 0, GH - 1)
    cx = gxc.astype(jnp.float32) * VS + X0 + VS / 2
    cy = gyc.astype(jnp.float32) * VS + Y0 + VS / 2
    dx = x - cx
    dy = y - cy
    z = jnp.zeros_like(x)
    aug = jnp.concatenate([pts, dx, dy, z, cx, cy, z], axis=0)
    h = jnp.dot(w1_ref[...], aug, preferred_element_type=jnp.float32) + b1_ref[...]
    h = jnp.maximum(h, 0.0)
    h = jnp.dot(w2_ref[...], h, preferred_element_type=jnp.float32) + b2_ref[...]
    h = jnp.maximum(h, 0.0)
    feat = jnp.clip(h, -100.0, 100.0)
    w = valid.astype(jnp.float32)
    out = jnp.concatenate(
        [feat * w, w, jnp.zeros((FW - C_OUT - 1, PT), jnp.float32)], axis=0)
    feat_ref[0] = out.T
    idx_ref[0] = jnp.where(valid, gyc * GW + gxc + off, BIG)


def _run_mlp(ptsT, W1f, b1f, W2f, b2f):
    return pl.pallas_call(
        _mlp_body,
        grid=(2 * NS,),
        in_specs=[
            pl.BlockSpec((1, 4, PT), lambda i: (i // NS, 0, i % NS)),
            pl.BlockSpec((C_OUT, 10), lambda i: (0, 0)),
            pl.BlockSpec((C_OUT, 1), lambda i: (0, 0)),
            pl.BlockSpec((C_OUT, C_OUT), lambda i: (0, 0)),
            pl.BlockSpec((C_OUT, 1), lambda i: (0, 0)),
        ],
        out_specs=[
            pl.BlockSpec((1, PT, FW), lambda i: (i // NS, i % NS, 0)),
            pl.BlockSpec((1, 1, PT), lambda i: (i, 0, 0)),
        ],
        out_shape=[
            jax.ShapeDtypeStruct((2, PPAD, FW), jnp.float32),
            jax.ShapeDtypeStruct((2 * NS, 1, PT), jnp.int32),
        ],
    )(ptsT, W1f, b1f, W2f, b2f)


def _make_scatter():
    mesh = plsc.VectorSubcoreMesh(core_axis_name="c", subcore_axis_name="s")

    @functools.partial(
        pl.kernel,
        mesh=mesh,
        out_type=jax.ShapeDtypeStruct((NZ * ZR, FW), jnp.float32),
        scratch_types=[
            pltpu.VMEM((PT,), jnp.int32),
            pltpu.VMEM((SELN,), jnp.int32),
            pltpu.VMEM((SELN,), jnp.int32),
            pltpu.VMEM((MAXG + 1, G), jnp.int32),
            pltpu.VMEM((G, FW), jnp.float32),
            pltpu.VMEM((G, FW), jnp.float32),
            pltpu.VMEM_SHARED((ACC_R, FW), jnp.float32),
            pltpu.SemaphoreType.DMA,
        ],
    )
    def scat(idx_hbm, feat_hbm, out_hbm,
             idx_v, selp_v, selr_v, selr2_v, rows_v, zb_v, acc, sem):
        c = lax.axis_index("c")
        s = lax.axis_index("s")
        lanes = lax.iota(jnp.int32, 16)
        zvec = jnp.zeros((16,), jnp.float32)

        def zb_body(r, carry):
            for k in range(FW // 16):
                zb_v[r, pl.ds(k * 16, 16)] = zvec
            return carry

        lax.fori_loop(0, G, zb_body, 0)

        for smp in range(2):
            pbase = smp * PPAD + s * PT
            pltpu.sync_copy(idx_hbm.at[pl.ds(pbase, PT)], idx_v)
            for j in range(NZ // NC // 2):
                zone = smp * 16 + c * 8 + j
                base = zone * ZR
                for r in range(ACC_R // NS // G):
                    pltpu.sync_copy(zb_v, acc.at[pl.ds(s * 1152 + r * G, G)])
                plsc.subcore_barrier()

                def cbody(i, cnt):
                    v = idx_v[pl.ds(i * 16, 16)]
                    m = (v >= base) & (v < base + ZR)
                    mi = m.astype(jnp.int32)
                    posv = cnt + plsc.cumsum(mi) - mi
                    pid = (pbase + i * 16) + lanes
                    plsc.store_scatter(selp_v, [posv], pid, mask=m)
                    plsc.store_scatter(selr_v, [posv], v - base, mask=m)
                    return cnt + jnp.sum(mi)

                cnt = lax.fori_loop(0, CHUNKS, cbody, 0)
                ng = (cnt + G - 1) // G

                def pbody(k, carry):
                    posv = cnt + k * 16 + lanes
                    plsc.store_scatter(selp_v, [posv], jnp.zeros((16,), jnp.int32))
                    plsc.store_scatter(selr_v, [posv], jnp.full((16,), DUMP, jnp.int32))
                    return carry

                lax.fori_loop(0, (ng * G - cnt + 15) // 16, pbody, 0)

                def gbody(g, carry):
                    for kk in range(G // 16):
                        selr2_v[g, pl.ds(kk * 16, 16)] = selr_v[pl.ds(g * G + kk * 16, 16)]
                    pltpu.async_copy(
                        feat_hbm.at[selp_v.at[pl.ds(g * G, G)]], rows_v, sem).wait()
                    pltpu.sync_copy(rows_v, acc.at[selr2_v.at[g]], add=True)
                    return carry

                lax.fori_loop(0, ng, gbody, 0)
                plsc.subcore_barrier()
                pltpu.sync_copy(acc.at[pl.ds(s * 1024, 1024)],
                                out_hbm.at[pl.ds(base + s * 1024, 1024)])
                plsc.subcore_barrier()

    return scat


_scatter = _make_scatter()


def _div_transpose_body(grid_ref, out_ref):
    blk = grid_ref[0]
    feat = blk[:, :, 0:C_OUT]
    cnt = blk[:, :, C_OUT:C_OUT + 1]
    r = feat / (cnt + 1e-6)
    out_ref[0] = jnp.transpose(r, (2, 0, 1))


def _run_div_transpose(grid):
    return pl.pallas_call(
        _div_transpose_body,
        grid=(2, GH // 8),
        in_specs=[pl.BlockSpec((1, 8, GW, FW), lambda sM, rM: (sM, rM, 0, 0))],
        out_specs=pl.BlockSpec((1, C_OUT, 8, GW), lambda sM, rM: (sM, 0, rM, 0)),
        out_shape=jax.ShapeDtypeStruct((2, C_OUT, GH, GW), jnp.float32),
    )(grid)


def kernel(points_list, W1, b1, g1, beta1, W2, b2, g2, beta2):
    s1 = g1 / jnp.sqrt(1.0 + EPS)
    s2 = g2 / jnp.sqrt(1.0 + EPS)
    W1f = W1 * s1[:, None]
    b1f = (b1 * s1 + beta1)[:, None]
    W2f = W2 * s2[:, None]
    b2f = (b2 * s2 + beta2)[:, None]
    ptsT = jnp.transpose(points_list, (0, 2, 1))
    ptsT = jnp.pad(ptsT, ((0, 0), (0, 0), (0, PPAD - NPTS)))
    feat, idx = _run_mlp(ptsT, W1f, b1f, W2f, b2f)
    featf = feat.reshape(2 * PPAD, FW)
    idxf = idx.reshape(2 * PPAD)
    grid = _scatter(idxf, featf)
    gridr = grid.reshape(2, GH, GW, FW)
    return _run_div_transpose(gridr)


# speedup vs baseline: 1.4734x; 1.4734x over previous
import functools

import jax
import jax.numpy as jnp
from jax import lax
from jax.experimental import pallas as pl
from jax.experimental.pallas import tpu as pltpu
from jax.experimental.pallas import tpu_sc as plsc

X0 = -51.2
Y0 = -51.2
VS = 0.2
GH = 512
GW = 512
C_OUT = 64
EPS = 1e-3

NPTS = 100000
NS = 16
NC = 2
PT = 6272
PPAD = NS * PT
FW = 128
ZR = 8192
NZS = 32
NZ = 2 * NZS
STRIPE = 544
ACC_R = NS * STRIPE
DUMP = ZR
BIG = 1 << 29
G = 128
CHUNKS = PT // 16
QG = 25
BMW = 16 * QG
SELN = PT + G


def _mlp_body(pts_ref, w1_ref, b1_ref, w2_ref, b2_ref, feat_ref, idx_ref):
    pid = pl.program_id(0)
    off = jnp.where(pid >= NS, GH * GW, 0)
    pts = pts_ref[0]
    x = pts[0:1, :]
    y = pts[1:2, :]
    gx = ((x - X0) / VS).astype(jnp.int32)
    gy = ((y - Y0) / VS).astype(jnp.int32)
    valid = (gx >= 0) & (gx < GW) & (gy >= 0) & (gy < GH)
    pos = lax.broadcasted_iota(jnp.int32, (1, PT), 1) + (pid % NS) * PT
    valid = valid & (pos < NPTS)
    gxc = jnp.clip(gx, 0, GW - 1)
    gyc = jnp.clip(gy, 0, GH - 1)
    cx = gxc.astype(jnp.float32) * VS + X0 + VS / 2
    cy = gyc.astype(jnp.float32) * VS + Y0 + VS / 2
    dx = x - cx
    dy = y - cy
    z = jnp.zeros_like(x)
    aug = jnp.concatenate([pts, dx, dy, z, cx, cy, z], axis=0)
    h = jnp.dot(w1_ref[...], aug, preferred_element_type=jnp.float32) + b1_ref[...]
    h = jnp.maximum(h, 0.0)
    h = jnp.dot(w2_ref[...], h, preferred_element_type=jnp.float32) + b2_ref[...]
    h = jnp.maximum(h, 0.0)
    feat = jnp.clip(h, -100.0, 100.0)
    w = valid.astype(jnp.float32)
    out = jnp.concatenate(
        [feat * w, w, jnp.zeros((FW - C_OUT - 1, PT), jnp.float32)], axis=0)
    feat_ref[0] = out.T
    pillar = gyc * GW + gxc
    idx_ref[0] = jnp.where(valid, pillar + off, BIG)


def _run_mlp(ptsT, W1f, b1f, W2f, b2f):
    return pl.pallas_call(
        _mlp_body,
        grid=(2 * NS,),
        in_specs=[
            pl.BlockSpec((1, 4, PT), lambda i: (i // NS, 0, i % NS)),
            pl.BlockSpec((C_OUT, 10), lambda i: (0, 0)),
            pl.BlockSpec((C_OUT, 1), lambda i: (0, 0)),
            pl.BlockSpec((C_OUT, C_OUT), lambda i: (0, 0)),
            pl.BlockSpec((C_OUT, 1), lambda i: (0, 0)),
        ],
        out_specs=[
            pl.BlockSpec((1, PT, FW), lambda i: (i // NS, i % NS, 0)),
            pl.BlockSpec((1, 1, PT), lambda i: (i, 0, 0)),
        ],
        out_shape=[
            jax.ShapeDtypeStruct((2, PPAD, FW), jnp.float32),
            jax.ShapeDtypeStruct((2 * NS, 1, PT), jnp.int32),
        ],
    )(ptsT, W1f, b1f, W2f, b2f)


def _make_scatter():
    mesh = plsc.VectorSubcoreMesh(core_axis_name="c", subcore_axis_name="s")

    @functools.partial(
        pl.kernel,
        mesh=mesh,
        out_type=jax.ShapeDtypeStruct((NZ * ZR, FW), jnp.float32),
        scratch_types=[
            pltpu.VMEM((PT,), jnp.int32),
            pltpu.VMEM((64,), jnp.int32),
            pltpu.VMEM((SELN,), jnp.int32),
            pltpu.VMEM((SELN,), jnp.int32),
            pltpu.VMEM((SELN // G, G), jnp.int32),
            pltpu.VMEM((G, FW), jnp.float32),
            pltpu.VMEM((G, FW), jnp.float32),
            pltpu.VMEM_SHARED((ACC_R, FW), jnp.float32),
            pltpu.SMEM((CHUNKS + 8,), jnp.int32),
            pltpu.SemaphoreType.DMA,
        ],
    )
    def scat(idx_hbm, feat_hbm, out_hbm,
             idx_v, util_v, selp_v, selr_v, selr2_v, rows_v, zb_v,
             acc, cnt_s, sem):
        c = lax.axis_index("c")
        s = lax.axis_index("s")
        lanes = lax.iota(jnp.int32, 16)
        zvec = jnp.zeros((16,), jnp.float32)

        def zb_body(r, carry):
            for k in range(FW // 16):
                zb_v[r, pl.ds(k * 16, 16)] = zvec
            return carry

        lax.fori_loop(0, G, zb_body, 0)

        for smp in range(2):
            pbase = smp * PPAD + s * PT
            pltpu.sync_copy(idx_hbm.at[pl.ds(pbase, PT)], idx_v)
            end_smp = (smp + 1) * GH * GW

            def zmbody(ci, carry):
                v = idx_v[pl.ds(ci * 16, 16)]
                zm = 0
                for k in range(16):
                    vk = v[k]
                    zk = (vk >> 13) & 31
                    zm = zm | jnp.where(vk < end_smp,
                                        lax.shift_left(1, zk), 0)
                cnt_s[ci] = zm
                return carry

            lax.fori_loop(0, CHUNKS, zmbody, 0)

            def zone_body(j, zcarry):
                zsam = c * 16 + j
                zone = smp * NZS + zsam
                base = zone * ZR
                for r in range(STRIPE // G):
                    pltpu.sync_copy(zb_v, acc.at[pl.ds(s * STRIPE + r * G, G)])
                pltpu.sync_copy(
                    zb_v.at[pl.ds(0, STRIPE % G)],
                    acc.at[pl.ds(s * STRIPE + (STRIPE // G) * G, STRIPE % G)])
                plsc.subcore_barrier()

                util_v[pl.ds(0, 16)] = jnp.full((16,), base, jnp.int32)
                cnt_s[CHUNKS] = 0

                def cbody(ci, carry):
                    zm = cnt_s[ci]

                    @pl.when(((zm >> zsam) & 1) != 0)
                    def _():
                        v = idx_v[pl.ds(ci * 16, 16)]
                        basev = util_v[pl.ds(0, 16)]
                        t = v - basev
                        m = plsc.bitcast(t, jnp.uint32) < jnp.uint32(ZR)
                        rowsv = jnp.where(m, t, DUMP)
                        util_v[pl.ds(48, 16)] = jnp.full(
                            (16,), pbase + ci * 16, jnp.int32)
                        pv = util_v[pl.ds(48, 16)]
                        wo = cnt_s[CHUNKS]
                        selr_v[pl.ds(wo, 16)] = rowsv
                        selp_v[pl.ds(wo, 16)] = pv + lanes
                        cnt_s[CHUNKS] = wo + 16
                    return carry

                lax.fori_loop(0, CHUNKS, cbody, 0)
                wo = cnt_s[CHUNKS]
                ng = (wo + G - 1) // G

                def padbody(p2, carry):
                    selr_v[pl.ds(wo + p2 * 16, 16)] = jnp.full((16,), DUMP, jnp.int32)
                    selp_v[pl.ds(wo + p2 * 16, 16)] = jnp.zeros((16,), jnp.int32)
                    return carry

                lax.fori_loop(0, (ng * G - wo) // 16, padbody, 0)

                def gbody(g2, carry):
                    for kk in range(G // 16):
                        selr2_v[g2, pl.ds(kk * 16, 16)] = (
                            selr_v[pl.ds(g2 * G + kk * 16, 16)])
                    pltpu.async_copy(
                        feat_hbm.at[selp_v.at[pl.ds(g2 * G, G)]], rows_v, sem).wait()
                    pltpu.sync_copy(rows_v, acc.at[selr2_v.at[g2]], add=True)
                    return carry

                lax.fori_loop(0, ng, gbody, 0)
                plsc.subcore_barrier()
                pltpu.sync_copy(acc.at[pl.ds(s * (ZR // NS), ZR // NS)],
                                out_hbm.at[pl.ds(base + s * (ZR // NS), ZR // NS)])
                plsc.subcore_barrier()
                return zcarry

            lax.fori_loop(0, NZS // NC, zone_body, 0)

    return scat


@functools.lru_cache(maxsize=1)
def _get_scatter():
    return _make_scatter()


def _div_transpose_body(grid_ref, out_ref):
    blk = grid_ref[0]
    feat = blk[:, :, 0:C_OUT]
    cnt = blk[:, :, C_OUT:C_OUT + 1]
    r = feat / (cnt + 1e-6)
    out_ref[0] = jnp.transpose(r, (2, 0, 1))


def _run_div_transpose(grid):
    return pl.pallas_call(
        _div_transpose_body,
        grid=(2, GH // 8),
        in_specs=[pl.BlockSpec((1, 8, GW, FW), lambda sM, rM: (sM, rM, 0, 0))],
        out_specs=pl.BlockSpec((1, C_OUT, 8, GW), lambda sM, rM: (sM, 0, rM, 0)),
        out_shape=jax.ShapeDtypeStruct((2, C_OUT, GH, GW), jnp.float32),
    )(grid)


def kernel(points_list, W1, b1, g1, beta1, W2, b2, g2, beta2):
    s1 = g1 / jnp.sqrt(1.0 + EPS)
    s2 = g2 / jnp.sqrt(1.0 + EPS)
    W1f = W1 * s1[:, None]
    b1f = (b1 * s1 + beta1)[:, None]
    W2f = W2 * s2[:, None]
    b2f = (b2 * s2 + beta2)[:, None]
    ptsT = jnp.transpose(points_list, (0, 2, 1))
    ptsT = jnp.pad(ptsT, ((0, 0), (0, 0), (0, PPAD - NPTS)))
    feat, idx = _run_mlp(ptsT, W1f, b1f, W2f, b2f)
    featf = feat.reshape(2 * PPAD, FW)
    idxf = idx.reshape(2 * PPAD)
    grid = _get_scatter()(idxf, featf)
    gridr = grid.reshape(2, GH, GW, FW)
    return _run_div_transpose(gridr)


# speedup vs baseline: 1.6604x; 1.1269x over previous
import functools

import jax
import jax.numpy as jnp
from jax import lax
from jax.experimental import pallas as pl
from jax.experimental.pallas import tpu as pltpu
from jax.experimental.pallas import tpu_sc as plsc

X0 = -51.2
Y0 = -51.2
VS = 0.2
GH = 512
GW = 512
C_OUT = 64
EPS = 1e-3

NPTS = 100000
NS = 16
NC = 2
PT = 6272
PPAD = NS * PT
FW = 128
ZR = 8192
NZS = 32
NZ = 2 * NZS
STRIPE = 544
ACC_R = NS * STRIPE
DUMP = ZR
BIG = 1 << 29
G = 128
CHUNKS = PT // 16
QG = 25
CH16 = 16 * QG
SELN = PT
NB = 2


def _mlp_body(pts_ref, w1_ref, b1_ref, w2_ref, b2_ref, feat_ref, idx_ref):
    pid = pl.program_id(0)
    off = jnp.where(pid >= NS, GH * GW, 0)
    pts = pts_ref[0]
    x = pts[0:1, :]
    y = pts[1:2, :]
    gx = ((x - X0) / VS).astype(jnp.int32)
    gy = ((y - Y0) / VS).astype(jnp.int32)
    valid = (gx >= 0) & (gx < GW) & (gy >= 0) & (gy < GH)
    pos = lax.broadcasted_iota(jnp.int32, (1, PT), 1) + (pid % NS) * PT
    valid = valid & (pos < NPTS)
    gxc = jnp.clip(gx, 0, GW - 1)
    gyc = jnp.clip(gy, 0, GH - 1)
    cx = gxc.astype(jnp.float32) * VS + X0 + VS / 2
    cy = gyc.astype(jnp.float32) * VS + Y0 + VS / 2
    dx = x - cx
    dy = y - cy
    z = jnp.zeros_like(x)
    aug = jnp.concatenate([pts, dx, dy, z, cx, cy, z], axis=0)
    h = jnp.dot(w1_ref[...], aug, preferred_element_type=jnp.float32) + b1_ref[...]
    h = jnp.maximum(h, 0.0)
    h = jnp.dot(w2_ref[...], h, preferred_element_type=jnp.float32) + b2_ref[...]
    h = jnp.maximum(h, 0.0)
    feat = jnp.clip(h, -100.0, 100.0)
    w = valid.astype(jnp.float32)
    out = jnp.concatenate(
        [feat * w, w, jnp.zeros((FW - C_OUT - 1, PT), jnp.float32)], axis=0)
    feat_ref[0] = out.T
    pillar = gyc * GW + gxc
    idx_ref[0] = jnp.where(valid, pillar + off, BIG)


def _run_mlp(ptsT, W1f, b1f, W2f, b2f):
    return pl.pallas_call(
        _mlp_body,
        grid=(2 * NS,),
        in_specs=[
            pl.BlockSpec((1, 4, PT), lambda i: (i // NS, 0, i % NS)),
            pl.BlockSpec((C_OUT, 10), lambda i: (0, 0)),
            pl.BlockSpec((C_OUT, 1), lambda i: (0, 0)),
            pl.BlockSpec((C_OUT, C_OUT), lambda i: (0, 0)),
            pl.BlockSpec((C_OUT, 1), lambda i: (0, 0)),
        ],
        out_specs=[
            pl.BlockSpec((1, PT, FW), lambda i: (i // NS, i % NS, 0)),
            pl.BlockSpec((1, 1, PT), lambda i: (i, 0, 0)),
        ],
        out_shape=[
            jax.ShapeDtypeStruct((2, PPAD, FW), jnp.float32),
            jax.ShapeDtypeStruct((2 * NS, 1, PT), jnp.int32),
        ],
    )(ptsT, W1f, b1f, W2f, b2f)


def _make_scatter():
    mesh = plsc.VectorSubcoreMesh(core_axis_name="c", subcore_axis_name="s")

    @functools.partial(
        pl.kernel,
        mesh=mesh,
        out_type=jax.ShapeDtypeStruct((NZ * ZR, FW), jnp.float32),
        scratch_types=[
            pltpu.VMEM((16 * CH16,), jnp.int32),
            pltpu.VMEM((CH16,), jnp.int32),
            pltpu.VMEM((64,), jnp.int32),
            pltpu.VMEM((SELN,), jnp.int32),
            pltpu.VMEM((SELN // G, G), jnp.int32),
            pltpu.VMEM((NB, G, FW), jnp.float32),
            pltpu.VMEM((32, FW), jnp.float32),
            pltpu.VMEM_SHARED((ACC_R, FW), jnp.float32),
            pltpu.SMEM((40,), jnp.int32),
            pltpu.SemaphoreType.DMA,
            pltpu.SemaphoreType.DMA,
            pltpu.SemaphoreType.DMA,
        ],
    )
    def scat(idx_hbm, feat_hbm, out_hbm,
             idx_v, zm_v, util_v, selp_v, selr2_v, rows_v, zb_v,
             acc, cnt_s, sem, sem2, semz):
        c = lax.axis_index("c")
        s = lax.axis_index("s")
        lanes = lax.iota(jnp.int32, 16)
        zvec = jnp.zeros((16,), jnp.float32)
        P8 = jnp.arange(16, dtype=jnp.int32) ^ 8
        P4 = jnp.arange(16, dtype=jnp.int32) ^ 4
        P2 = jnp.arange(16, dtype=jnp.int32) ^ 2
        P1 = jnp.arange(16, dtype=jnp.int32) ^ 1

        def zb_body(r, carry):
            for k in range(FW // 16):
                zb_v[r, pl.ds(k * 16, 16)] = zvec
            return carry

        lax.fori_loop(0, 32, zb_body, 0)

        for smp in range(2):
            pbase = smp * PPAD + s * PT
            pltpu.sync_copy(idx_hbm.at[pl.ds(pbase, PT)], idx_v.at[pl.ds(0, PT)])
            for k in range(16 * CH16 // 16 - PT // 16):
                idx_v[pl.ds(PT + k * 16, 16)] = jnp.full((16,), BIG, jnp.int32)
            end_smp = (smp + 1) * GH * GW

            def zmgroup(q, carry):
                pack = jnp.zeros((16,), jnp.int32)
                for k in range(16):
                    ci = q * 16 + k
                    v = idx_v[pl.ds(ci * 16, 16)]
                    bits = jnp.where(v < end_smp,
                                     jnp.left_shift(1, (v >> 13) & 31), 0)
                    bits = bits | bits[P8]
                    bits = bits | bits[P4]
                    bits = bits | bits[P2]
                    bits = bits | bits[P1]
                    pack = jnp.where(lanes == k, bits, pack)
                zm_v[pl.ds(q * 16, 16)] = pack
                gor = pack | pack[P8]
                gor = gor | gor[P4]
                gor = gor | gor[P2]
                gor = gor | gor[P1]
                util_v[pl.ds(0, 16)] = gor
                gl = util_v[pl.ds(0, 16)]
                cnt_s[q] = gl[0]
                return carry

            lax.fori_loop(0, QG, zmgroup, 0)

            def zone_body(j, zcarry):
                zsam = c * 16 + j
                zone = smp * NZS + zsam
                base = zone * ZR
                zd = []
                for r in range(STRIPE // 32):
                    zd.append(pltpu.async_copy(
                        zb_v, acc.at[pl.ds(s * STRIPE + r * 32, 32)], semz))
                for d in zd:
                    d.wait()
                plsc.subcore_barrier()

                util_v[pl.ds(0, 16)] = jnp.full((16,), base, jnp.int32)
                cnt_s[32] = 0

                def cgroup(q, carry):
                    gm = cnt_s[q]

                    @pl.when(((gm >> zsam) & 1) != 0)
                    def _():
                        zmvec = zm_v[pl.ds(q * 16, 16)]
                        for k in range(16):
                            zmk = zmvec[k]

                            @pl.when(((zmk >> zsam) & 1) != 0)
                            def _():
                                ci = q * 16 + k
                                v = idx_v[pl.ds(ci * 16, 16)]
                                basev = util_v[pl.ds(0, 16)]
                                t = v - basev
                                m = plsc.bitcast(t, jnp.uint32) < jnp.uint32(ZR)
                                rowsv = jnp.where(m, t, DUMP)
                                util_v[pl.ds(48, 16)] = jnp.full(
                                    (16,), pbase + ci * 16, jnp.int32)
                                pv = util_v[pl.ds(48, 16)]
                                wo = cnt_s[32]
                                selr2_v[wo >> 7, pl.ds(wo & 127, 16)] = rowsv
                                selp_v[pl.ds(wo, 16)] = pv + lanes
                                cnt_s[32] = wo + 16
                    return carry

                lax.fori_loop(0, QG, cgroup, 0)
                wo = cnt_s[32]
                ng = (wo + G - 1) // G

                def padbody(p2, carry):
                    po = wo + p2 * 16
                    selr2_v[po >> 7, pl.ds(po & 127, 16)] = jnp.full(
                        (16,), DUMP, jnp.int32)
                    selp_v[pl.ds(po, 16)] = jnp.zeros((16,), jnp.int32)
                    return carry

                lax.fori_loop(0, (ng * G - wo) // 16, padbody, 0)

                def super_body(p2, carry):
                    for b in range(NB):
                        g2 = p2 * NB + b

                        @pl.when(g2 < ng)
                        def _():
                            pltpu.async_copy(
                                feat_hbm.at[selp_v.at[pl.ds(g2 * G, G)]],
                                rows_v.at[b], sem)
                    for b in range(NB):
                        g2 = p2 * NB + b

                        @pl.when(g2 < ng)
                        def _():
                            pltpu.make_async_copy(
                                feat_hbm.at[selp_v.at[pl.ds(g2 * G, G)]],
                                rows_v.at[b], sem).wait()
                    for b in range(NB):
                        g2 = p2 * NB + b

                        @pl.when(g2 < ng)
                        def _():
                            pltpu.async_copy(
                                rows_v.at[b], acc.at[selr2_v.at[g2]], sem2,
                                add=True)
                    for b in range(NB):
                        g2 = p2 * NB + b

                        @pl.when(g2 < ng)
                        def _():
                            pltpu.make_async_copy(
                                rows_v.at[b], acc.at[selr2_v.at[g2]], sem2).wait()
                    return carry

                lax.fori_loop(0, (ng + NB - 1) // NB, super_body, 0)
                plsc.subcore_barrier()
                pltpu.sync_copy(acc.at[pl.ds(s * (ZR // NS), ZR // NS)],
                                out_hbm.at[pl.ds(base + s * (ZR // NS), ZR // NS)])
                plsc.subcore_barrier()
                return zcarry

            lax.fori_loop(0, NZS // NC, zone_body, 0)

    return scat


@functools.lru_cache(maxsize=1)
def _get_scatter():
    return _make_scatter()


def _div_transpose_body(grid_ref, out_ref):
    blk = grid_ref[0]
    feat = blk[:, :, 0:C_OUT]
    cnt = blk[:, :, C_OUT:C_OUT + 1]
    r = feat / (cnt + 1e-6)
    out_ref[0] = jnp.transpose(r, (2, 0, 1))


def _run_div_transpose(grid):
    return pl.pallas_call(
        _div_transpose_body,
        grid=(2, GH // 8),
        in_specs=[pl.BlockSpec((1, 8, GW, FW), lambda sM, rM: (sM, rM, 0, 0))],
        out_specs=pl.BlockSpec((1, C_OUT, 8, GW), lambda sM, rM: (sM, 0, rM, 0)),
        out_shape=jax.ShapeDtypeStruct((2, C_OUT, GH, GW), jnp.float32),
    )(grid)


def kernel(points_list, W1, b1, g1, beta1, W2, b2, g2, beta2):
    s1 = g1 / jnp.sqrt(1.0 + EPS)
    s2 = g2 / jnp.sqrt(1.0 + EPS)
    W1f = W1 * s1[:, None]
    b1f = (b1 * s1 + beta1)[:, None]
    W2f = W2 * s2[:, None]
    b2f = (b2 * s2 + beta2)[:, None]
    ptsT = jnp.transpose(points_list, (0, 2, 1))
    ptsT = jnp.pad(ptsT, ((0, 0), (0, 0), (0, PPAD - NPTS)))
    feat, idx = _run_mlp(ptsT, W1f, b1f, W2f, b2f)
    featf = feat.reshape(2 * PPAD, FW)
    idxf = idx.reshape(2 * PPAD)
    grid = _get_scatter()(idxf, featf)
    gridr = grid.reshape(2, GH, GW, FW)
    return _run_div_transpose(gridr)


# speedup vs baseline: 2.0197x; 1.2164x over previous
import functools

import jax
import jax.numpy as jnp
from jax import lax
from jax.experimental import pallas as pl
from jax.experimental.pallas import tpu as pltpu
from jax.experimental.pallas import tpu_sc as plsc

X0 = -51.2
Y0 = -51.2
VS = 0.2
GH = 512
GW = 512
C_OUT = 64
EPS = 1e-3

NPTS = 100000
NS = 16
NC = 2
PT = 6272
PPAD = NS * PT
FW = 128
ZR = 8192
NZS = 32
NZ = 2 * NZS
STRIPE = 544
ACC_R = NS * STRIPE
DUMP = ZR
BIG = 1 << 29
G = 128
CHUNKS = PT // 16
QG = 25
CH16 = 16 * QG
SELN = PT
NB = 2


def _mlp_body(pts_ref, w1_ref, b1_ref, w2_ref, b2_ref, feat_ref, idx_ref):
    pid = pl.program_id(0)
    off = jnp.where(pid >= NS, GH * GW, 0)
    pts = pts_ref[0]
    x = pts[0:1, :]
    y = pts[1:2, :]
    gx = ((x - X0) / VS).astype(jnp.int32)
    gy = ((y - Y0) / VS).astype(jnp.int32)
    valid = (gx >= 0) & (gx < GW) & (gy >= 0) & (gy < GH)
    pos = lax.broadcasted_iota(jnp.int32, (1, PT), 1) + (pid % NS) * PT
    valid = valid & (pos < NPTS)
    gxc = jnp.clip(gx, 0, GW - 1)
    gyc = jnp.clip(gy, 0, GH - 1)
    cx = gxc.astype(jnp.float32) * VS + X0 + VS / 2
    cy = gyc.astype(jnp.float32) * VS + Y0 + VS / 2
    dx = x - cx
    dy = y - cy
    z = jnp.zeros_like(x)
    aug = jnp.concatenate([pts, dx, dy, z, cx, cy, z], axis=0)
    h = jnp.dot(w1_ref[...], aug, preferred_element_type=jnp.float32) + b1_ref[...]
    h = jnp.maximum(h, 0.0)
    h = jnp.dot(w2_ref[...], h, preferred_element_type=jnp.float32) + b2_ref[...]
    h = jnp.maximum(h, 0.0)
    feat = jnp.clip(h, -100.0, 100.0)
    w = valid.astype(jnp.float32)
    out = jnp.concatenate(
        [feat * w, w, jnp.zeros((FW - C_OUT - 1, PT), jnp.float32)], axis=0)
    feat_ref[0] = out.T
    pillar = gyc * GW + gxc
    idx_ref[0] = jnp.where(valid, pillar + off, BIG)


def _run_mlp(ptsT, W1f, b1f, W2f, b2f):
    return pl.pallas_call(
        _mlp_body,
        grid=(2 * NS,),
        in_specs=[
            pl.BlockSpec((1, 4, PT), lambda i: (i // NS, 0, i % NS)),
            pl.BlockSpec((C_OUT, 10), lambda i: (0, 0)),
            pl.BlockSpec((C_OUT, 1), lambda i: (0, 0)),
            pl.BlockSpec((C_OUT, C_OUT), lambda i: (0, 0)),
            pl.BlockSpec((C_OUT, 1), lambda i: (0, 0)),
        ],
        out_specs=[
            pl.BlockSpec((1, PT, FW), lambda i: (i // NS, i % NS, 0)),
            pl.BlockSpec((1, 1, PT), lambda i: (i, 0, 0)),
        ],
        out_shape=[
            jax.ShapeDtypeStruct((2, PPAD, FW), jnp.float32),
            jax.ShapeDtypeStruct((2 * NS, 1, PT), jnp.int32),
        ],
    )(ptsT, W1f, b1f, W2f, b2f)


def _make_scatter():
    mesh = plsc.VectorSubcoreMesh(core_axis_name="c", subcore_axis_name="s")

    @functools.partial(
        pl.kernel,
        mesh=mesh,
        out_type=jax.ShapeDtypeStruct((NZ * ZR, FW), jnp.float32),
        scratch_types=[
            pltpu.VMEM((16 * CH16,), jnp.int32),
            pltpu.VMEM((CH16,), jnp.int32),
            pltpu.VMEM((64,), jnp.int32),
            pltpu.VMEM((SELN,), jnp.int32),
            pltpu.VMEM((SELN // G, G), jnp.int32),
            pltpu.VMEM((NB, G, FW), jnp.float32),
            pltpu.VMEM((32, FW), jnp.float32),
            pltpu.VMEM_SHARED((ACC_R, FW), jnp.float32),
            pltpu.SMEM((40,), jnp.int32),
            pltpu.SemaphoreType.DMA,
            pltpu.SemaphoreType.DMA,
            pltpu.SemaphoreType.DMA,
        ],
    )
    def scat(idx_hbm, feat_hbm, out_hbm,
             idx_v, zm_v, util_v, selp_v, selr2_v, rows_v, zb_v,
             acc, cnt_s, sem, sem2, semz):
        c = lax.axis_index("c")
        s = lax.axis_index("s")
        lanes = lax.iota(jnp.int32, 16)
        zvec = jnp.zeros((16,), jnp.float32)
        P8 = jnp.arange(16, dtype=jnp.int32) ^ 8
        P4 = jnp.arange(16, dtype=jnp.int32) ^ 4
        P2 = jnp.arange(16, dtype=jnp.int32) ^ 2
        P1 = jnp.arange(16, dtype=jnp.int32) ^ 1

        def zb_body(r, carry):
            for k in range(FW // 16):
                zb_v[r, pl.ds(k * 16, 16)] = zvec
            return carry

        lax.fori_loop(0, 32, zb_body, 0)
        cnt_s[33] = 0
        plsc.subcore_barrier()

        for smp in range(2):
            pbase = smp * PPAD + s * PT
            pltpu.sync_copy(idx_hbm.at[pl.ds(pbase, PT)], idx_v.at[pl.ds(0, PT)])
            for k in range(16 * CH16 // 16 - PT // 16):
                idx_v[pl.ds(PT + k * 16, 16)] = jnp.full((16,), BIG, jnp.int32)
            end_smp = (smp + 1) * GH * GW

            def zmgroup(q, carry):
                pack = jnp.zeros((16,), jnp.int32)
                for k in range(16):
                    ci = q * 16 + k
                    v = idx_v[pl.ds(ci * 16, 16)]
                    bits = jnp.where(v < end_smp,
                                     jnp.left_shift(1, (v >> 13) & 31), 0)
                    bits = bits | bits[P8]
                    bits = bits | bits[P4]
                    bits = bits | bits[P2]
                    bits = bits | bits[P1]
                    pack = jnp.where(lanes == k, bits, pack)
                zm_v[pl.ds(q * 16, 16)] = pack
                gor = pack | pack[P8]
                gor = gor | gor[P4]
                gor = gor | gor[P2]
                gor = gor | gor[P1]
                util_v[pl.ds(0, 16)] = gor
                gl = util_v[pl.ds(0, 16)]
                cnt_s[q] = gl[0]
                return carry

            lax.fori_loop(0, QG, zmgroup, 0)

            def zone_body(j, zcarry):
                zsam = c * 16 + j
                zone = smp * NZS + zsam
                base = zone * ZR

                util_v[pl.ds(0, 16)] = jnp.full((16,), base, jnp.int32)
                cnt_s[32] = 0

                def cgroup(q, carry):
                    gm = cnt_s[q]

                    @pl.when(((gm >> zsam) & 1) != 0)
                    def _():
                        zmvec = zm_v[pl.ds(q * 16, 16)]
                        for k in range(16):
                            zmk = zmvec[k]

                            @pl.when(((zmk >> zsam) & 1) != 0)
                            def _():
                                ci = q * 16 + k
                                v = idx_v[pl.ds(ci * 16, 16)]
                                basev = util_v[pl.ds(0, 16)]
                                t = v - basev
                                m = plsc.bitcast(t, jnp.uint32) < jnp.uint32(ZR)
                                rowsv = jnp.where(m, t, DUMP)
                                util_v[pl.ds(48, 16)] = jnp.full(
                                    (16,), pbase + ci * 16, jnp.int32)
                                pv = util_v[pl.ds(48, 16)]
                                wo = cnt_s[32]
                                selr2_v[wo >> 7, pl.ds(wo & 127, 16)] = rowsv
                                selp_v[pl.ds(wo, 16)] = pv + lanes
                                cnt_s[32] = wo + 16
                    return carry

                lax.fori_loop(0, QG, cgroup, 0)
                wo = cnt_s[32]

                plsc.fetch_and_add(cnt_s.at[33], wo, subcore_id=0)
                plsc.subcore_barrier()
                tot = plsc.fetch_and_add(cnt_s.at[33], 0, subcore_id=0)
                plsc.subcore_barrier()

                @pl.when(s == 0)
                def _():
                    cnt_s[33] = 0

                @pl.when(tot != 0)
                def _():
                    zd = []
                    for r in range(STRIPE // 32):
                        zd.append(pltpu.async_copy(
                            zb_v, acc.at[pl.ds(s * STRIPE + r * 32, 32)], semz))
                    for d in zd:
                        d.wait()
                    plsc.subcore_barrier()
                    ng = (wo + G - 1) // G

                    def padbody(p2, carry):
                        po = wo + p2 * 16
                        selr2_v[po >> 7, pl.ds(po & 127, 16)] = jnp.full(
                            (16,), DUMP, jnp.int32)
                        selp_v[pl.ds(po, 16)] = jnp.zeros((16,), jnp.int32)
                        return carry

                    lax.fori_loop(0, (ng * G - wo) // 16, padbody, 0)

                    def super_body(p2, carry):
                        for b in range(NB):
                            g2 = p2 * NB + b

                            @pl.when(g2 < ng)
                            def _():
                                pltpu.async_copy(
                                    feat_hbm.at[selp_v.at[pl.ds(g2 * G, G)]],
                                    rows_v.at[b], sem)
                        for b in range(NB):
                            g2 = p2 * NB + b

                            @pl.when(g2 < ng)
                            def _():
                                pltpu.make_async_copy(
                                    feat_hbm.at[selp_v.at[pl.ds(g2 * G, G)]],
                                    rows_v.at[b], sem).wait()
                        for b in range(NB):
                            g2 = p2 * NB + b

                            @pl.when(g2 < ng)
                            def _():
                                pltpu.async_copy(
                                    rows_v.at[b], acc.at[selr2_v.at[g2]], sem2,
                                    add=True)
                        for b in range(NB):
                            g2 = p2 * NB + b

                            @pl.when(g2 < ng)
                            def _():
                                pltpu.make_async_copy(
                                    rows_v.at[b], acc.at[selr2_v.at[g2]],
                                    sem2).wait()
                        return carry

                    lax.fori_loop(0, (ng + NB - 1) // NB, super_body, 0)
                    plsc.subcore_barrier()
                    pltpu.sync_copy(
                        acc.at[pl.ds(s * (ZR // NS), ZR // NS)],
                        out_hbm.at[pl.ds(base + s * (ZR // NS), ZR // NS)])

                @pl.when(tot == 0)
                def _():
                    zd = []
                    for r in range(ZR // NS // 32):
                        zd.append(pltpu.async_copy(
                            zb_v,
                            out_hbm.at[pl.ds(base + s * (ZR // NS) + r * 32, 32)],
                            semz))
                    for d in zd:
                        d.wait()
                plsc.subcore_barrier()
                return zcarry

            lax.fori_loop(0, NZS // NC, zone_body, 0)

    return scat


@functools.lru_cache(maxsize=1)
def _get_scatter():
    return _make_scatter()


def _div_transpose_body(grid_ref, out_ref):
    blk = grid_ref[0]
    feat = blk[:, :, 0:C_OUT]
    cnt = blk[:, :, C_OUT:C_OUT + 1]
    r = feat / (cnt + 1e-6)
    out_ref[0] = jnp.transpose(r, (2, 0, 1))


def _run_div_transpose(grid):
    return pl.pallas_call(
        _div_transpose_body,
        grid=(2, GH // 8),
        in_specs=[pl.BlockSpec((1, 8, GW, FW), lambda sM, rM: (sM, rM, 0, 0))],
        out_specs=pl.BlockSpec((1, C_OUT, 8, GW), lambda sM, rM: (sM, 0, rM, 0)),
        out_shape=jax.ShapeDtypeStruct((2, C_OUT, GH, GW), jnp.float32),
    )(grid)


def kernel(points_list, W1, b1, g1, beta1, W2, b2, g2, beta2):
    s1 = g1 / jnp.sqrt(1.0 + EPS)
    s2 = g2 / jnp.sqrt(1.0 + EPS)
    W1f = W1 * s1[:, None]
    b1f = (b1 * s1 + beta1)[:, None]
    W2f = W2 * s2[:, None]
    b2f = (b2 * s2 + beta2)[:, None]
    ptsT = jnp.transpose(points_list, (0, 2, 1))
    ptsT = jnp.pad(ptsT, ((0, 0), (0, 0), (0, PPAD - NPTS)))
    feat, idx = _run_mlp(ptsT, W1f, b1f, W2f, b2f)
    featf = feat.reshape(2 * PPAD, FW)
    idxf = idx.reshape(2 * PPAD)
    grid = _get_scatter()(idxf, featf)
    gridr = grid.reshape(2, GH, GW, FW)
    return _run_div_transpose(gridr)
